# Initial kernel scaffold; baseline (speedup 1.0000x reference)
import functools

import jax
import jax.numpy as jnp
from jax import lax
from jax.experimental import pallas as pl
from jax.experimental.pallas import tpu as pltpu
from jax.experimental.pallas import tpu_sc as plsc

N, E, D = 10000, 320000, 128
NC, NS, L = 2, 16, 16
NW = NC * NS
EPW = E // NW
K = 128
CH = -(-EPW // K)
EPW_P = CH * K
JROW = N
ZSTRIPE = 632
N_SH = NS * ZSTRIPE
OSTRIPE = N // NS

_sc_mesh = plsc.VectorSubcoreMesh(core_axis_name="c", subcore_axis_name="s")


def _sc_agg_body(sup_hbm, srcp_hbm, dstp_hbm, wp_hbm, out_hbm,
                 src_v, dst_v, w_v, rows_v, acc_sh, sem):
    cid = lax.axis_index("c")
    sid = lax.axis_index("s")
    wid = sid * NC + cid

    pltpu.sync_copy(srcp_hbm.at[wid], src_v)
    pltpu.sync_copy(dstp_hbm.at[wid], dst_v)
    pltpu.sync_copy(wp_hbm.at[wid], w_v)

    def _zrow(r, _):
        for j in range(D // L):
            rows_v[r, pl.ds(j * L, L)] = jnp.zeros((L,), jnp.float32)
        return 0
    lax.fori_loop(0, K, _zrow, 0)
    zbase = sid * ZSTRIPE
    for t in range(ZSTRIPE // K):
        pltpu.sync_copy(rows_v, acc_sh.at[pl.ds(zbase + t * K, K)])
    rem = ZSTRIPE % K
    if rem:
        pltpu.sync_copy(rows_v.at[pl.ds(0, rem)],
                        acc_sh.at[pl.ds(zbase + (ZSTRIPE // K) * K, rem)])
    plsc.subcore_barrier()

    def _chunk(c, _):
        pltpu.async_copy(sup_hbm.at[src_v.at[c]], rows_v, sem).wait()

        def _edge(e, _):
            w = w_v[c, e]
            for j in range(D // L):
                sl = pl.ds(j * L, L)
                rows_v[e, sl] = rows_v[e, sl] * w
            return 0
        lax.fori_loop(0, K, _edge, 0)

        pltpu.sync_copy(rows_v, acc_sh.at[dst_v.at[c]], add=True)
        return 0
    lax.fori_loop(0, CH, _chunk, 0)
    plsc.subcore_barrier()

    ob = sid * OSTRIPE
    pltpu.sync_copy(acc_sh.at[pl.ds(ob, OSTRIPE)],
                    out_hbm.at[cid, pl.ds(ob, OSTRIPE)])


_sc_agg = pl.kernel(
    _sc_agg_body,
    out_type=jax.ShapeDtypeStruct((NC, N, D), jnp.float32),
    mesh=_sc_mesh,
    scratch_types=[
        pltpu.VMEM((CH, K), jnp.int32),
        pltpu.VMEM((CH, K), jnp.int32),
        pltpu.VMEM((CH, K), jnp.float32),
        pltpu.VMEM((K, D), jnp.float32),
        pltpu.VMEM_SHARED((N_SH, D), jnp.float32),
        pltpu.SemaphoreType.DMA,
    ],
)


_BM = 2000


def _mm1_body(x_ref, w_ref, o_ref):
    o_ref[...] = jnp.dot(x_ref[...], w_ref[...],
                         preferred_element_type=jnp.float32)


def _mm2_body(p_ref, b_ref, w_ref, o_ref):
    h = jnp.maximum(p_ref[0] + p_ref[1] + b_ref[...], 0.0)
    o_ref[...] = jnp.dot(h, w_ref[...], preferred_element_type=jnp.float32)


def _fin_body(p_ref, b_ref, o_ref):
    o_ref[...] = p_ref[0] + p_ref[1] + b_ref[...]


def _mm1(x, w):
    return pl.pallas_call(
        _mm1_body,
        grid=(N // _BM,),
        in_specs=[
            pl.BlockSpec((_BM, D), lambda i: (i, 0)),
            pl.BlockSpec((D, D), lambda i: (0, 0)),
        ],
        out_specs=pl.BlockSpec((_BM, D), lambda i: (i, 0)),
        out_shape=jax.ShapeDtypeStruct((N, D), jnp.float32),
    )(x, w)


def _mm2(p, b, w):
    return pl.pallas_call(
        _mm2_body,
        grid=(N // _BM,),
        in_specs=[
            pl.BlockSpec((NC, _BM, D), lambda i: (0, i, 0)),
            pl.BlockSpec((1, D), lambda i: (0, 0)),
            pl.BlockSpec((D, D), lambda i: (0, 0)),
        ],
        out_specs=pl.BlockSpec((_BM, D), lambda i: (i, 0)),
        out_shape=jax.ShapeDtypeStruct((N, D), jnp.float32),
    )(p, b.reshape(1, D), w)


def _fin(p, b):
    return pl.pallas_call(
        _fin_body,
        grid=(N // _BM,),
        in_specs=[
            pl.BlockSpec((NC, _BM, D), lambda i: (0, i, 0)),
            pl.BlockSpec((1, D), lambda i: (0, 0)),
        ],
        out_specs=pl.BlockSpec((_BM, D), lambda i: (i, 0)),
        out_shape=jax.ShapeDtypeStruct((N, D), jnp.float32),
    )(p, b.reshape(1, D))


def kernel(features, edge_index, edge_weight, W1, b1, W2, b2):
    pad = EPW_P - EPW
    src = edge_index[0].reshape(NW, EPW)
    dst = edge_index[1].reshape(NW, EPW)
    ew = edge_weight.reshape(NW, EPW)
    srcp = jnp.pad(src, ((0, 0), (0, pad))).reshape(NW, CH, K)
    dstp = jnp.pad(dst, ((0, 0), (0, pad)),
                   constant_values=JROW).reshape(NW, CH, K)
    wp = jnp.pad(ew, ((0, 0), (0, pad))).reshape(NW, CH, K)

    sup1 = _mm1(features, W1)
    p1 = _sc_agg(sup1, srcp, dstp, wp)
    sup2 = _mm2(p1, b1, W2)
    p2 = _sc_agg(sup2, srcp, dstp, wp)
    return _fin(p2, b2)


# speedup vs baseline: 4.2676x; 4.2676x over previous
import jax
import jax.numpy as jnp
from jax import lax
from jax.experimental import pallas as pl
from jax.experimental.pallas import tpu as pltpu
from jax.experimental.pallas import tpu_sc as plsc

N, E, D = 10000, 320000, 128
NC, NS, L = 2, 16, 16
NW = NC * NS
EPW = E // NW
K = 128
CH = -(-EPW // K)
EPW_P = CH * K
JROW = N
ZSTRIPE = 632
N_SH = NS * ZSTRIPE

_sc_mesh = plsc.VectorSubcoreMesh(core_axis_name="c", subcore_axis_name="s",
                                  num_cores=NC, num_subcores=NS)


def _sc_agg_body(sup_hbm, srcp_hbm, dstp_hbm, wp_hbm, out_hbm,
                 src_v, dst_v, w_v, rows_v, acc_sh, sem):
    cid = lax.axis_index("c")
    sid = lax.axis_index("s")
    wid = sid * NC + cid

    pltpu.sync_copy(srcp_hbm.at[wid], src_v)
    pltpu.sync_copy(dstp_hbm.at[wid], dst_v)
    pltpu.sync_copy(wp_hbm.at[wid], w_v)

    def _zrow(r, _):
        for j in range(D // L):
            rows_v[r, pl.ds(j * L, L)] = jnp.zeros((L,), jnp.float32)
        return 0
    lax.fori_loop(0, K, _zrow, 0)
    zbase = sid * ZSTRIPE
    for t in range(ZSTRIPE // K):
        pltpu.sync_copy(rows_v, acc_sh.at[pl.ds(zbase + t * K, K)])
    rem = ZSTRIPE % K
    if rem:
        pltpu.sync_copy(rows_v.at[pl.ds(0, rem)],
                        acc_sh.at[pl.ds(zbase + (ZSTRIPE // K) * K, rem)])
    plsc.subcore_barrier()

    def _chunk(c, _):
        pltpu.async_copy(sup_hbm.at[src_v.at[c]], rows_v, sem).wait()

        def _edge(e, _):
            wv = plsc.load_gather(w_v, [jnp.full((L,), c * K + e, jnp.int32)])
            for j in range(D // L):
                sl = pl.ds(j * L, L)
                rows_v[e, sl] = rows_v[e, sl] * wv
            return 0
        lax.fori_loop(0, K, _edge, 0)

        pltpu.sync_copy(rows_v, acc_sh.at[dst_v.at[c]], add=True)
        return 0
    lax.fori_loop(0, CH, _chunk, 0)
    plsc.subcore_barrier()

    pltpu.sync_copy(acc_sh.at[pl.ds(zbase, ZSTRIPE)],
                    out_hbm.at[cid, pl.ds(zbase, ZSTRIPE)])


_sc_agg = pl.kernel(
    _sc_agg_body,
    out_type=jax.ShapeDtypeStruct((NC, N_SH, D), jnp.float32),
    mesh=_sc_mesh,
    scratch_types=[
        pltpu.VMEM((CH, K), jnp.int32),
        pltpu.VMEM((CH, K), jnp.int32),
        pltpu.VMEM((CH * K,), jnp.float32),
        pltpu.VMEM((K, D), jnp.float32),
        pltpu.VMEM_SHARED((N_SH, D), jnp.float32),
        pltpu.SemaphoreType.DMA,
    ],
    compiler_params=pltpu.CompilerParams(needs_layout_passes=False),
)


_BM = 2000


def _mm1_body(x_ref, w_ref, o_ref):
    o_ref[...] = jnp.dot(x_ref[...], w_ref[...],
                         preferred_element_type=jnp.float32)


def _mm2_body(p_ref, b_ref, w_ref, o_ref):
    h = jnp.maximum(p_ref[0] + p_ref[1] + b_ref[...], 0.0)
    o_ref[...] = jnp.dot(h, w_ref[...], preferred_element_type=jnp.float32)


def _fin_body(p_ref, b_ref, o_ref):
    o_ref[...] = p_ref[0] + p_ref[1] + b_ref[...]


def _mm1(x, w):
    return pl.pallas_call(
        _mm1_body,
        grid=(N // _BM,),
        in_specs=[
            pl.BlockSpec((_BM, D), lambda i: (i, 0)),
            pl.BlockSpec((D, D), lambda i: (0, 0)),
        ],
        out_specs=pl.BlockSpec((_BM, D), lambda i: (i, 0)),
        out_shape=jax.ShapeDtypeStruct((N, D), jnp.float32),
    )(x, w)


def _mm2(p, b, w):
    return pl.pallas_call(
        _mm2_body,
        grid=(N // _BM,),
        in_specs=[
            pl.BlockSpec((NC, _BM, D), lambda i: (0, i, 0)),
            pl.BlockSpec((1, D), lambda i: (0, 0)),
            pl.BlockSpec((D, D), lambda i: (0, 0)),
        ],
        out_specs=pl.BlockSpec((_BM, D), lambda i: (i, 0)),
        out_shape=jax.ShapeDtypeStruct((N, D), jnp.float32),
    )(p, b.reshape(1, D), w)


def _fin(p, b):
    return pl.pallas_call(
        _fin_body,
        grid=(N // _BM,),
        in_specs=[
            pl.BlockSpec((NC, _BM, D), lambda i: (0, i, 0)),
            pl.BlockSpec((1, D), lambda i: (0, 0)),
        ],
        out_specs=pl.BlockSpec((_BM, D), lambda i: (i, 0)),
        out_shape=jax.ShapeDtypeStruct((N, D), jnp.float32),
    )(p, b.reshape(1, D))


def kernel(features, edge_index, edge_weight, W1, b1, W2, b2):
    pad = EPW_P - EPW
    src = edge_index[0].reshape(NW, EPW)
    dst = edge_index[1].reshape(NW, EPW)
    ew = edge_weight.reshape(NW, EPW)
    srcp = jnp.pad(src, ((0, 0), (0, pad))).reshape(NW, CH, K)
    dstp = jnp.pad(dst, ((0, 0), (0, pad)),
                   constant_values=JROW).reshape(NW, CH, K)
    wp = jnp.pad(ew, ((0, 0), (0, pad)))

    sup1 = _mm1(features, W1)
    p1 = _sc_agg(sup1, srcp, dstp, wp)
    sup2 = _mm2(p1, b1, W2)
    p2 = _sc_agg(sup2, srcp, dstp, wp)
    return _fin(p2, b2)


# speedup vs baseline: 5.7038x; 1.3365x over previous
import jax
import jax.numpy as jnp
from jax import lax
from jax.experimental import pallas as pl
from jax.experimental.pallas import tpu as pltpu
from jax.experimental.pallas import tpu_sc as plsc

N, E, D = 10000, 320000, 128
NC, NS, L = 2, 16, 16
DH = D // NC
EPT = E // NS
K = 128
NBUF = 3
CH = NBUF * (-(-EPT // (K * NBUF)))
EPT_P = CH * K
JROW = N
ZSTRIPE = 632
N_SH = NS * ZSTRIPE

_sc_mesh = plsc.VectorSubcoreMesh(core_axis_name="c", subcore_axis_name="s",
                                  num_cores=NC, num_subcores=NS)


def _sc_agg_body(sup_hbm, srcp_hbm, dstp_hbm, wp_hbm, out_hbm,
                 src_v, dst_v, w_v, rows0, rows1, rows2, acc_sh,
                 gs0, gs1, gs2, ss0, ss1, ss2):
    rows = (rows0, rows1, rows2)
    gsem = (gs0, gs1, gs2)
    ssem = (ss0, ss1, ss2)
    cid = lax.axis_index("c")
    sid = lax.axis_index("s")
    sup = sup_hbm.at[cid]

    pltpu.sync_copy(srcp_hbm.at[sid], src_v)
    pltpu.sync_copy(dstp_hbm.at[sid], dst_v)
    pltpu.sync_copy(wp_hbm.at[sid], w_v)

    def _zrow(r, _):
        for j in range(DH // L):
            rows0[r, pl.ds(j * L, L)] = jnp.zeros((L,), jnp.float32)
        return 0
    lax.fori_loop(0, K, _zrow, 0)
    zbase = sid * ZSTRIPE
    for t in range(ZSTRIPE // K):
        pltpu.sync_copy(rows0, acc_sh.at[pl.ds(zbase + t * K, K)])
    rem = ZSTRIPE % K
    if rem:
        pltpu.sync_copy(rows0.at[pl.ds(0, rem)],
                        acc_sh.at[pl.ds(zbase + (ZSTRIPE // K) * K, rem)])
    plsc.subcore_barrier()

    def _gwait(b):
        pltpu.make_async_copy(sup_hbm.at[0, pl.ds(0, K)], rows[b],
                              gsem[b]).wait()

    def _swait(b):
        pltpu.make_async_copy(sup_hbm.at[0, pl.ds(0, K)], rows[b],
                              ssem[b]).wait()

    def _scale(b, m):
        def _edge(e, _):
            wv = plsc.load_gather(w_v, [jnp.full((L,), m * K + e, jnp.int32)])
            for j in range(DH // L):
                sl = pl.ds(j * L, L)
                rows[b][e, sl] = rows[b][e, sl] * wv
            return 0
        lax.fori_loop(0, K, _edge, 0, unroll=4)

    for b in range(NBUF):
        pltpu.async_copy(sup.at[src_v.at[b]], rows[b], gsem[b])

    for b in range(NBUF):
        _gwait(b)
        _scale(b, b)
        pltpu.async_copy(rows[b], acc_sh.at[dst_v.at[b]], ssem[b], add=True)
        if b > 0:
            _swait(b - 1)
            pltpu.async_copy(sup.at[src_v.at[b + 2]], rows[b - 1],
                             gsem[b - 1])

    def _mchunk(mm, _):
        for b in range(NBUF):
            m = mm * NBUF + b
            bp = (b + NBUF - 1) % NBUF
            _gwait(b)
            _scale(b, m)
            pltpu.async_copy(rows[b], acc_sh.at[dst_v.at[m]], ssem[b],
                             add=True)
            _swait(bp)
            pltpu.async_copy(sup.at[src_v.at[m + 2]], rows[bp], gsem[bp])
        return 0
    lax.fori_loop(1, CH // NBUF - 1, _mchunk, 0)

    for b in range(NBUF):
        m = CH - NBUF + b
        _gwait(b)
        _scale(b, m)
        pltpu.async_copy(rows[b], acc_sh.at[dst_v.at[m]], ssem[b], add=True)
        if b == 0:
            _swait(NBUF - 1)
            pltpu.async_copy(sup.at[src_v.at[CH - 1]], rows[NBUF - 1],
                             gsem[NBUF - 1])
    for b in range(NBUF):
        _swait(b)
    plsc.subcore_barrier()

    pltpu.sync_copy(acc_sh.at[pl.ds(zbase, ZSTRIPE)],
                    out_hbm.at[cid, pl.ds(zbase, ZSTRIPE)])


_sc_agg = pl.kernel(
    _sc_agg_body,
    out_type=jax.ShapeDtypeStruct((NC, N_SH, DH), jnp.float32),
    mesh=_sc_mesh,
    scratch_types=[
        pltpu.VMEM((CH, K), jnp.int32),
        pltpu.VMEM((CH, K), jnp.int32),
        pltpu.VMEM((CH * K,), jnp.float32),
        pltpu.VMEM((K, DH), jnp.float32),
        pltpu.VMEM((K, DH), jnp.float32),
        pltpu.VMEM((K, DH), jnp.float32),
        pltpu.VMEM_SHARED((N_SH, DH), jnp.float32),
        pltpu.SemaphoreType.DMA,
        pltpu.SemaphoreType.DMA,
        pltpu.SemaphoreType.DMA,
        pltpu.SemaphoreType.DMA,
        pltpu.SemaphoreType.DMA,
        pltpu.SemaphoreType.DMA,
    ],
    compiler_params=pltpu.CompilerParams(needs_layout_passes=False,
                                         use_tc_tiling_on_sc=False),
)


_BM = 2000


def _mm1_body(x_ref, w_ref, o_ref):
    s = jnp.dot(x_ref[...], w_ref[...], preferred_element_type=jnp.float32)
    o_ref[0] = s[:, :DH]
    o_ref[1] = s[:, DH:]


def _mm2_body(p_ref, b_ref, w_ref, o_ref):
    h0 = jnp.maximum(p_ref[0] + b_ref[0, :DH], 0.0)
    h1 = jnp.maximum(p_ref[1] + b_ref[0, DH:], 0.0)
    s = (jnp.dot(h0, w_ref[:DH, :], preferred_element_type=jnp.float32)
         + jnp.dot(h1, w_ref[DH:, :], preferred_element_type=jnp.float32))
    o_ref[0] = s[:, :DH]
    o_ref[1] = s[:, DH:]


def _fin_body(p_ref, b_ref, o_ref):
    o_ref[:, :DH] = p_ref[0] + b_ref[0, :DH]
    o_ref[:, DH:] = p_ref[1] + b_ref[0, DH:]


def _mm1(x, w):
    return pl.pallas_call(
        _mm1_body,
        grid=(N // _BM,),
        in_specs=[
            pl.BlockSpec((_BM, D), lambda i: (i, 0)),
            pl.BlockSpec((D, D), lambda i: (0, 0)),
        ],
        out_specs=pl.BlockSpec((NC, _BM, DH), lambda i: (0, i, 0)),
        out_shape=jax.ShapeDtypeStruct((NC, N, DH), jnp.float32),
    )(x, w)


def _mm2(p, b, w):
    return pl.pallas_call(
        _mm2_body,
        grid=(N // _BM,),
        in_specs=[
            pl.BlockSpec((NC, _BM, DH), lambda i: (0, i, 0)),
            pl.BlockSpec((1, D), lambda i: (0, 0)),
            pl.BlockSpec((D, D), lambda i: (0, 0)),
        ],
        out_specs=pl.BlockSpec((NC, _BM, DH), lambda i: (0, i, 0)),
        out_shape=jax.ShapeDtypeStruct((NC, N, DH), jnp.float32),
    )(p, b.reshape(1, D), w)


def _fin(p, b):
    return pl.pallas_call(
        _fin_body,
        grid=(N // _BM,),
        in_specs=[
            pl.BlockSpec((NC, _BM, DH), lambda i: (0, i, 0)),
            pl.BlockSpec((1, D), lambda i: (0, 0)),
        ],
        out_specs=pl.BlockSpec((_BM, D), lambda i: (i, 0)),
        out_shape=jax.ShapeDtypeStruct((N, D), jnp.float32),
    )(p, b.reshape(1, D))


def kernel(features, edge_index, edge_weight, W1, b1, W2, b2):
    pad = EPT_P - EPT
    src = edge_index[0].reshape(NS, EPT)
    dst = edge_index[1].reshape(NS, EPT)
    ew = edge_weight.reshape(NS, EPT)
    srcp = jnp.pad(src, ((0, 0), (0, pad))).reshape(NS, CH, K)
    dstp = jnp.pad(dst, ((0, 0), (0, pad)),
                   constant_values=JROW).reshape(NS, CH, K)
    wp = jnp.pad(ew, ((0, 0), (0, pad)))

    sup1 = _mm1(features, W1)
    agg1 = _sc_agg(sup1, srcp, dstp, wp)
    sup2 = _mm2(agg1[:, :N], b1, W2)
    agg2 = _sc_agg(sup2, srcp, dstp, wp)
    return _fin(agg2[:, :N], b2)
